# Initial kernel scaffold; baseline (speedup 1.0000x reference)
import jax
import jax.numpy as jnp
from jax.experimental import pallas as pl


def kernel(x, edge_index, W, b, gamma, beta):
    raise NotImplementedError("write your pallas kernel here")


# speedup vs baseline: 9.7931x; 9.7931x over previous
import functools

import jax
import jax.numpy as jnp
from jax import lax
from jax.experimental import pallas as pl
from jax.experimental.pallas import tpu as pltpu
from jax.experimental.pallas import tpu_sc as plsc

N = 10000
E = 320000
D = 128

NC = 2
NS = 16
NW = NC * NS
C = 64
STEPS = 160
QS = 40
EPT = C * STEPS
E_PAD = EPT * NW
N_PAD = 10240
RPT = N_PAD // NS

_mesh = plsc.VectorSubcoreMesh(core_axis_name="c", subcore_axis_name="s")


@functools.partial(
    pl.kernel,
    out_type=jax.ShapeDtypeStruct((NC, N_PAD), jnp.float32),
    mesh=_mesh,
    scratch_types=[
        pltpu.VMEM((STEPS, C), jnp.int32),
        pltpu.VMEM((C,), jnp.float32),
        pltpu.VMEM((C,), jnp.float32),
        pltpu.VMEM_SHARED((N_PAD,), jnp.float32),
    ],
)
def _deg_kernel(col_hbm, out_hbm, col_v, ones_v, zeros_v, deg_sh):
    cid = lax.axis_index("c")
    sid = lax.axis_index("s")
    wid = cid * NS + sid
    for i in range(C // 16):
        ones_v[pl.ds(i * 16, 16)] = jnp.ones((16,), jnp.float32)
        zeros_v[pl.ds(i * 16, 16)] = jnp.zeros((16,), jnp.float32)
    for j in range(RPT // C):
        pltpu.sync_copy(zeros_v, deg_sh.at[pl.ds(sid * RPT + j * C, C)])
    plsc.subcore_barrier()
    pltpu.sync_copy(col_hbm.at[wid], col_v)

    def body(step, _):
        pltpu.sync_copy(ones_v, deg_sh.at[col_v.at[step]], add=True)
        return _

    lax.fori_loop(0, STEPS, body, None)
    plsc.subcore_barrier()
    pltpu.sync_copy(deg_sh.at[pl.ds(sid * RPT, RPT)],
                    out_hbm.at[cid, pl.ds(sid * RPT, RPT)])


@functools.partial(
    pl.kernel,
    out_type=jax.ShapeDtypeStruct((NC, N_PAD, D), jnp.float32),
    mesh=_mesh,
    scratch_types=[
        pltpu.VMEM((QS, C), jnp.int32),
        pltpu.VMEM((QS, C), jnp.int32),
        pltpu.VMEM((C, D), jnp.float32),
        pltpu.VMEM((C, D), jnp.float32),
        pltpu.VMEM((16, D), jnp.float32),
        pltpu.VMEM_SHARED((N_PAD, D), jnp.float32),
        pltpu.SemaphoreType.DMA,
        pltpu.SemaphoreType.DMA,
    ],
)
def _agg_kernel(s_hbm, row_hbm, col_hbm, out_hbm,
                row_v, col_v, g_a, g_b, zb, agg_sh, sem_a, sem_b):
    cid = lax.axis_index("c")
    sid = lax.axis_index("s")
    wid = cid * NS + sid
    for i in range(16):
        for j in range(D // 16):
            zb[i, pl.ds(j * 16, 16)] = jnp.zeros((16,), jnp.float32)

    def zbody(j, _):
        pltpu.sync_copy(zb, agg_sh.at[pl.ds(sid * RPT + j * 16, 16)])
        return _

    lax.fori_loop(0, RPT // 16, zbody, None)
    plsc.subcore_barrier()

    def body(i, _):
        pltpu.make_async_copy(s_hbm.at[row_v.at[2 * i + 1]], g_b, sem_b).start()
        pltpu.make_async_copy(s_hbm.at[row_v.at[2 * i]], g_a, sem_a).wait()
        pltpu.sync_copy(g_a, agg_sh.at[col_v.at[2 * i]], add=True)

        @pl.when(i < QS // 2 - 1)
        def _():
            pltpu.make_async_copy(s_hbm.at[row_v.at[2 * i + 2]], g_a,
                                  sem_a).start()

        pltpu.make_async_copy(s_hbm.at[row_v.at[2 * i + 1]], g_b, sem_b).wait()
        pltpu.sync_copy(g_b, agg_sh.at[col_v.at[2 * i + 1]], add=True)
        return _

    for q in range(STEPS // QS):
        pltpu.sync_copy(row_hbm.at[wid, pl.ds(q * QS, QS)], row_v)
        pltpu.sync_copy(col_hbm.at[wid, pl.ds(q * QS, QS)], col_v)
        pltpu.make_async_copy(s_hbm.at[row_v.at[0]], g_a, sem_a).start()
        lax.fori_loop(0, QS // 2, body, None)
    plsc.subcore_barrier()
    pltpu.sync_copy(agg_sh.at[pl.ds(sid * RPT, RPT)],
                    out_hbm.at[cid, pl.ds(sid * RPT, RPT)])


def _mm_body(x_ref, w_ref, deg_ref, s_ref):
    deg = deg_ref[0] + deg_ref[1]
    dis = jnp.where(deg > 0.0, lax.rsqrt(deg), 0.0)
    h = lax.dot_general(x_ref[...], w_ref[...],
                        (((1,), (1,)), ((), ())),
                        preferred_element_type=jnp.float32)
    s_ref[...] = h * dis


def _final_body(x_ref, agg_ref, deg_ref, b_ref, g_ref, beta_ref, o_ref):
    agg = agg_ref[0, :N, :] + agg_ref[1, :N, :]
    deg = deg_ref[0, :N, :] + deg_ref[1, :N, :]
    dis = jnp.where(deg > 0.0, lax.rsqrt(deg), 0.0)
    pre = agg * dis + b_ref[...]
    mean = jnp.mean(pre, axis=0, keepdims=True)
    cent = pre - mean
    var = jnp.mean(cent * cent, axis=0, keepdims=True)
    norm = cent * lax.rsqrt(var + 1e-5) * g_ref[...] + beta_ref[...]
    o_ref[...] = x_ref[...] + jnp.maximum(norm, 0.0)


def kernel(x, edge_index, W, b, gamma, beta):
    row = edge_index[0]
    col = edge_index[1]
    pad = jnp.full((E_PAD - E,), N, dtype=jnp.int32)
    row_r = jnp.concatenate([row, pad]).reshape(NW, STEPS, C)
    col_r = jnp.concatenate([col, pad]).reshape(NW, STEPS, C)
    x_pad = jnp.pad(x, ((0, N_PAD - N), (0, 0)))

    deg_pair = _deg_kernel(col_r)
    deg3 = deg_pair.reshape(NC, N_PAD, 1)

    s = pl.pallas_call(
        _mm_body,
        out_shape=jax.ShapeDtypeStruct((N_PAD, D), jnp.float32),
    )(x_pad, W, deg3)

    agg_pair = _agg_kernel(s, row_r, col_r)

    out = pl.pallas_call(
        _final_body,
        out_shape=jax.ShapeDtypeStruct((N, D), jnp.float32),
    )(x, agg_pair, deg3, b.reshape(1, D), gamma.reshape(1, D),
      beta.reshape(1, D))
    return out


# speedup vs baseline: 27.4209x; 2.8000x over previous
import functools

import jax
import jax.numpy as jnp
from jax import lax
from jax.experimental import pallas as pl
from jax.experimental.pallas import tpu as pltpu
from jax.experimental.pallas import tpu_sc as plsc

N = 10000
E = 320000
D = 128

NC = 2
NS = 16
NW = NC * NS
C = 64
STEPS = 160
QS = 40
EPT = C * STEPS
E_PAD = EPT * NW
N_PAD = 10240
RPT = N_PAD // NS

_mesh = plsc.VectorSubcoreMesh(core_axis_name="c", subcore_axis_name="s")


@functools.partial(
    pl.kernel,
    out_type=jax.ShapeDtypeStruct((NC, N_PAD), jnp.float32),
    mesh=_mesh,
    scratch_types=[
        pltpu.VMEM((STEPS, C), jnp.int32),
        pltpu.VMEM((C,), jnp.float32),
        pltpu.VMEM((C,), jnp.float32),
        pltpu.VMEM_SHARED((N_PAD,), jnp.float32),
    ],
)
def _deg_kernel(col_hbm, out_hbm, col_v, ones_v, zeros_v, deg_sh):
    cid = lax.axis_index("c")
    sid = lax.axis_index("s")
    wid = cid * NS + sid
    for i in range(C // 16):
        ones_v[pl.ds(i * 16, 16)] = jnp.ones((16,), jnp.float32)
        zeros_v[pl.ds(i * 16, 16)] = jnp.zeros((16,), jnp.float32)
    for j in range(RPT // C):
        pltpu.sync_copy(zeros_v, deg_sh.at[pl.ds(sid * RPT + j * C, C)])
    plsc.subcore_barrier()
    pltpu.sync_copy(col_hbm.at[wid], col_v)

    def body(step, _):
        pltpu.sync_copy(ones_v, deg_sh.at[col_v.at[step]], add=True)
        return _

    lax.fori_loop(0, STEPS, body, None)
    plsc.subcore_barrier()
    pltpu.sync_copy(deg_sh.at[pl.ds(sid * RPT, RPT)],
                    out_hbm.at[cid, pl.ds(sid * RPT, RPT)])


@functools.partial(
    pl.kernel,
    out_type=jax.ShapeDtypeStruct((NC, N_PAD, D), jnp.float32),
    mesh=_mesh,
    scratch_types=[
        pltpu.VMEM((QS, C), jnp.int32),
        pltpu.VMEM((QS, C), jnp.int32),
        pltpu.VMEM((C, D), jnp.float32),
        pltpu.VMEM((C, D), jnp.float32),
        pltpu.VMEM((16, D), jnp.float32),
        pltpu.VMEM_SHARED((N_PAD, D), jnp.float32),
        pltpu.SemaphoreType.DMA,
        pltpu.SemaphoreType.DMA,
    ],
)
def _agg_kernel(s_hbm, row_hbm, col_hbm, out_hbm,
                row_v, col_v, g_a, g_b, zb, agg_sh, sem_a, sem_b):
    cid = lax.axis_index("c")
    sid = lax.axis_index("s")
    wid = cid * NS + sid
    for i in range(16):
        for j in range(D // 16):
            zb[i, pl.ds(j * 16, 16)] = jnp.zeros((16,), jnp.float32)

    def zbody(j, _):
        pltpu.sync_copy(zb, agg_sh.at[pl.ds(sid * RPT + j * 16, 16)])
        return _

    lax.fori_loop(0, RPT // 16, zbody, None)
    plsc.subcore_barrier()

    def body(i, _):
        pltpu.make_async_copy(s_hbm.at[row_v.at[2 * i + 1]], g_b, sem_b).start()
        pltpu.make_async_copy(s_hbm.at[row_v.at[2 * i]], g_a, sem_a).wait()
        pltpu.sync_copy(g_a, agg_sh.at[col_v.at[2 * i]], add=True)

        @pl.when(i < QS // 2 - 1)
        def _():
            pltpu.make_async_copy(s_hbm.at[row_v.at[2 * i + 2]], g_a,
                                  sem_a).start()

        pltpu.make_async_copy(s_hbm.at[row_v.at[2 * i + 1]], g_b, sem_b).wait()
        pltpu.sync_copy(g_b, agg_sh.at[col_v.at[2 * i + 1]], add=True)
        return _

    for q in range(STEPS // QS):
        pltpu.sync_copy(row_hbm.at[wid, pl.ds(q * QS, QS)], row_v)
        pltpu.sync_copy(col_hbm.at[wid, pl.ds(q * QS, QS)], col_v)
        pltpu.make_async_copy(s_hbm.at[row_v.at[0]], g_a, sem_a).start()
        lax.fori_loop(0, QS // 2, body, None)
    plsc.subcore_barrier()
    pltpu.sync_copy(agg_sh.at[pl.ds(sid * RPT, RPT)],
                    out_hbm.at[cid, pl.ds(sid * RPT, RPT)])


def _mm_body(x_ref, w_ref, deg_ref, s_ref):
    deg = deg_ref[0] + deg_ref[1]
    dis = jnp.where(deg > 0.0, lax.rsqrt(deg), 0.0)
    h = lax.dot_general(x_ref[...], w_ref[...],
                        (((1,), (1,)), ((), ())),
                        preferred_element_type=jnp.float32)
    s_ref[...] = h * dis


def _final_body(x_ref, agg_ref, deg_ref, b_ref, g_ref, beta_ref, o_ref):
    agg = agg_ref[0, :N, :] + agg_ref[1, :N, :]
    deg = deg_ref[0, :N, :] + deg_ref[1, :N, :]
    dis = jnp.where(deg > 0.0, lax.rsqrt(deg), 0.0)
    pre = agg * dis + b_ref[...]
    mean = jnp.mean(pre, axis=0, keepdims=True)
    cent = pre - mean
    var = jnp.mean(cent * cent, axis=0, keepdims=True)
    norm = cent * lax.rsqrt(var + 1e-5) * g_ref[...] + beta_ref[...]
    o_ref[...] = x_ref[...] + jnp.maximum(norm, 0.0)


def kernel(x, edge_index, W, b, gamma, beta):
    row = edge_index[0]
    col = edge_index[1]
    pad = N + jnp.arange(E_PAD - E, dtype=jnp.int32) % (N_PAD - N)
    row_r = jnp.concatenate([row, pad]).reshape(NW, STEPS, C)
    col_r = jnp.concatenate([col, pad]).reshape(NW, STEPS, C)
    x_pad = jnp.pad(x, ((0, N_PAD - N), (0, 0)))

    deg_pair = _deg_kernel(col_r)
    deg3 = deg_pair.reshape(NC, N_PAD, 1)

    s = pl.pallas_call(
        _mm_body,
        out_shape=jax.ShapeDtypeStruct((N_PAD, D), jnp.float32),
    )(x_pad, W, deg3)

    agg_pair = _agg_kernel(s, row_r, col_r)

    out = pl.pallas_call(
        _final_body,
        out_shape=jax.ShapeDtypeStruct((N, D), jnp.float32),
    )(x, agg_pair, deg3, b.reshape(1, D), gamma.reshape(1, D),
      beta.reshape(1, D))
    return out


# speedup vs baseline: 31.3479x; 1.1432x over previous
import functools

import jax
import jax.numpy as jnp
from jax import lax
from jax.experimental import pallas as pl
from jax.experimental.pallas import tpu as pltpu
from jax.experimental.pallas import tpu_sc as plsc

N = 10000
E = 320000
D = 128

NC = 2
NS = 16
NW = NC * NS
C = 128
STEPS = 80
QS = 16
EPT = C * STEPS
E_PAD = EPT * NW
N_PAD = 10240
RPT = N_PAD // NS

_mesh = plsc.VectorSubcoreMesh(core_axis_name="c", subcore_axis_name="s")


@functools.partial(
    pl.kernel,
    out_type=jax.ShapeDtypeStruct((NC, N_PAD), jnp.float32),
    mesh=_mesh,
    scratch_types=[
        pltpu.VMEM((STEPS, C), jnp.int32),
        pltpu.VMEM((C,), jnp.float32),
        pltpu.VMEM((C,), jnp.float32),
        pltpu.VMEM_SHARED((N_PAD,), jnp.float32),
    ],
)
def _deg_kernel(col_hbm, out_hbm, col_v, ones_v, zeros_v, deg_sh):
    cid = lax.axis_index("c")
    sid = lax.axis_index("s")
    wid = cid * NS + sid
    for i in range(C // 16):
        ones_v[pl.ds(i * 16, 16)] = jnp.ones((16,), jnp.float32)
        zeros_v[pl.ds(i * 16, 16)] = jnp.zeros((16,), jnp.float32)
    for j in range(RPT // C):
        pltpu.sync_copy(zeros_v, deg_sh.at[pl.ds(sid * RPT + j * C, C)])
    plsc.subcore_barrier()
    pltpu.sync_copy(col_hbm.at[wid], col_v)

    def body(step, _):
        pltpu.sync_copy(ones_v, deg_sh.at[col_v.at[step]], add=True)
        return _

    lax.fori_loop(0, STEPS, body, None)
    plsc.subcore_barrier()
    pltpu.sync_copy(deg_sh.at[pl.ds(sid * RPT, RPT)],
                    out_hbm.at[cid, pl.ds(sid * RPT, RPT)])


@functools.partial(
    pl.kernel,
    out_type=jax.ShapeDtypeStruct((NC, N_PAD, D), jnp.float32),
    mesh=_mesh,
    scratch_types=[
        pltpu.VMEM((QS, C), jnp.int32),
        pltpu.VMEM((QS, C), jnp.int32),
        pltpu.VMEM((C, D), jnp.float32),
        pltpu.VMEM((C, D), jnp.float32),
        pltpu.VMEM_SHARED((N_PAD, D), jnp.float32),
        pltpu.SemaphoreType.DMA,
        pltpu.SemaphoreType.DMA,
    ],
)
def _agg_kernel(s_hbm, row_hbm, col_hbm, out_hbm,
                row_v, col_v, g_a, g_b, agg_sh, sem_a, sem_b):
    cid = lax.axis_index("c")
    sid = lax.axis_index("s")
    wid = cid * NS + sid

    def zrow(i, _):
        for j in range(D // 16):
            g_a[i, pl.ds(j * 16, 16)] = jnp.zeros((16,), jnp.float32)
        return _

    lax.fori_loop(0, C, zrow, None)

    def zbody(j, _):
        pltpu.sync_copy(g_a, agg_sh.at[pl.ds(sid * RPT + j * C, C)])
        return _

    lax.fori_loop(0, RPT // C, zbody, None)
    plsc.subcore_barrier()

    def body(i, _):
        pltpu.make_async_copy(s_hbm.at[row_v.at[2 * i + 1]], g_b, sem_b).start()
        pltpu.make_async_copy(s_hbm.at[row_v.at[2 * i]], g_a, sem_a).wait()
        pltpu.sync_copy(g_a, agg_sh.at[col_v.at[2 * i]], add=True)

        @pl.when(i < QS // 2 - 1)
        def _():
            pltpu.make_async_copy(s_hbm.at[row_v.at[2 * i + 2]], g_a,
                                  sem_a).start()

        pltpu.make_async_copy(s_hbm.at[row_v.at[2 * i + 1]], g_b, sem_b).wait()
        pltpu.sync_copy(g_b, agg_sh.at[col_v.at[2 * i + 1]], add=True)
        return _

    for q in range(STEPS // QS):
        pltpu.sync_copy(row_hbm.at[wid, pl.ds(q * QS, QS)], row_v)
        pltpu.sync_copy(col_hbm.at[wid, pl.ds(q * QS, QS)], col_v)
        pltpu.make_async_copy(s_hbm.at[row_v.at[0]], g_a, sem_a).start()
        lax.fori_loop(0, QS // 2, body, None)
    plsc.subcore_barrier()
    pltpu.sync_copy(agg_sh.at[pl.ds(sid * RPT, RPT)],
                    out_hbm.at[cid, pl.ds(sid * RPT, RPT)])


def _mm_body(x_ref, w_ref, deg_ref, s_ref):
    deg = deg_ref[0] + deg_ref[1]
    dis = jnp.where(deg > 0.0, lax.rsqrt(deg), 0.0)
    h = lax.dot_general(x_ref[...], w_ref[...],
                        (((1,), (1,)), ((), ())),
                        preferred_element_type=jnp.float32)
    s_ref[...] = h * dis


def _final_body(x_ref, agg_ref, deg_ref, b_ref, g_ref, beta_ref, o_ref):
    agg = agg_ref[0, :N, :] + agg_ref[1, :N, :]
    deg = deg_ref[0, :N, :] + deg_ref[1, :N, :]
    dis = jnp.where(deg > 0.0, lax.rsqrt(deg), 0.0)
    pre = agg * dis + b_ref[...]
    mean = jnp.mean(pre, axis=0, keepdims=True)
    cent = pre - mean
    var = jnp.mean(cent * cent, axis=0, keepdims=True)
    norm = cent * lax.rsqrt(var + 1e-5) * g_ref[...] + beta_ref[...]
    o_ref[...] = x_ref[...] + jnp.maximum(norm, 0.0)


def kernel(x, edge_index, W, b, gamma, beta):
    row = edge_index[0]
    col = edge_index[1]
    pad = N + jnp.arange(E_PAD - E, dtype=jnp.int32) % (N_PAD - N)
    row_r = jnp.concatenate([row, pad]).reshape(NW, STEPS, C)
    col_r = jnp.concatenate([col, pad]).reshape(NW, STEPS, C)
    x_pad = jnp.pad(x, ((0, N_PAD - N), (0, 0)))

    deg_pair = _deg_kernel(col_r)
    deg3 = deg_pair.reshape(NC, N_PAD, 1)

    s = pl.pallas_call(
        _mm_body,
        out_shape=jax.ShapeDtypeStruct((N_PAD, D), jnp.float32),
    )(x_pad, W, deg3)

    agg_pair = _agg_kernel(s, row_r, col_r)

    out = pl.pallas_call(
        _final_body,
        out_shape=jax.ShapeDtypeStruct((N, D), jnp.float32),
    )(x, agg_pair, deg3, b.reshape(1, D), gamma.reshape(1, D),
      beta.reshape(1, D))
    return out
